# baseline (speedup 1.0000x reference)
import functools

import jax
import jax.numpy as jnp
from jax import lax
from jax.experimental import pallas as pl
from jax.experimental.pallas import tpu as pltpu
from jax.experimental.pallas import tpu_sc as plsc

B = 16
D = 1024


def _extract_body(packed_hbm, len_hbm, uns_hbm, out_hbm, len_v, uns_v, idx_v,
                  rows_v, sem):
    c = lax.axis_index("c")
    s = lax.axis_index("s")

    @pl.when(jnp.logical_and(c == 0, s == 0))
    def _():
        pltpu.sync_copy(len_hbm, len_v)
        pltpu.sync_copy(uns_hbm, uns_v)
        lv = len_v[...]
        lm1 = lv - 1
        acc = uns_v[...]
        for j in range(B):
            bj = jnp.broadcast_to(lv[j], (B,))
            acc = acc + jnp.minimum(bj, lm1)
        idx_v[...] = acc
        pltpu.async_copy(packed_hbm.at[idx_v], rows_v, sem).wait()
        pltpu.sync_copy(rows_v, out_hbm)


@functools.partial(jax.jit, static_argnames=())
def _last_element_extract(packed_data, lengths_i32, unsorted_i32):
    mesh = plsc.VectorSubcoreMesh(core_axis_name="c", subcore_axis_name="s")
    fn = functools.partial(
        pl.kernel,
        mesh=mesh,
        out_type=jax.ShapeDtypeStruct((B, D), jnp.float32),
        scratch_types=[
            pltpu.VMEM((B,), jnp.int32),
            pltpu.VMEM((B,), jnp.int32),
            pltpu.VMEM((B,), jnp.int32),
            pltpu.VMEM((B, D), jnp.float32),
            pltpu.SemaphoreType.DMA,
        ],
    )(_extract_body)
    return fn(packed_data, lengths_i32, unsorted_i32)


def kernel(packed_data, batch_sizes, sorted_indices, unsorted_indices, lengths):
    del batch_sizes, sorted_indices
    return _last_element_extract(
        packed_data,
        lengths.astype(jnp.int32),
        unsorted_indices.astype(jnp.int32),
    )
